# baseline (speedup 1.0000x reference)
import functools

import jax
import jax.numpy as jnp
from jax import lax
from jax.experimental import pallas as pl
from jax.experimental.pallas import tpu as pltpu
from jax.experimental.pallas import tpu_sc as plsc

N = 50000
E = 800000
F0 = 100
FA = 104
NLAB = 19

NC = 2
NS = 16
C = 5120
NCHUNK = 10
NPAD = NCHUNK * C
EPT = 51200
EPAD = EPT * NS
EB = 2048
GB = 128
NEB = EPT // EB
NGB = EB // GB
STRIPE = C // NS


SBKT = 26
BKT = SBKT * EB
STG = 2560
KPC = NCHUNK // NC

_mesh = plsc.VectorSubcoreMesh(core_axis_name="c", subcore_axis_name="s")
_sc_params = pltpu.CompilerParams(use_tc_tiling_on_sc=False,
                                  needs_layout_passes=False)


def _partition_body(c, s, src_hbm, dst_hbm, bpack_hbm, bcnt_hbm,
                    ebs, ebd, stgs, cntbuf):
    e0 = s * EPT

    def eblk(b, carry):
        boff = e0 + b * EB
        pltpu.sync_copy(src_hbm.at[pl.ds(boff, EB)], ebs)
        pltpu.sync_copy(dst_hbm.at[pl.ds(boff, EB)], ebd)
        new = []
        for k in range(KPC):
            W, FB = carry[2 * k], carry[2 * k + 1]
            p = k * NC + c
            lo = p * C
            stg = stgs[k]

            def grp(j, w, stg=stg, lo=lo):
                sv = ebs[pl.ds(j * 16, 16)]
                dv = ebd[pl.ds(j * 16, 16)]
                ld = dv - lo
                m = (ld >= 0) & (ld < C)
                packed = sv * 8192 + jnp.where(m, ld, 0)
                _ = plsc.sort_key_val(m.astype(jnp.int32), packed,
                                      descending=True)
                stg[pl.ds(w, 16)] = _[1]
                return w + plsc.all_reduce_population_count(m)[0]

            W2 = lax.fori_loop(0, EB // 16, grp, W, unroll=4)
            nfull = W2 // GB

            def flush(rb, _, stg=stg, p=p, FB=FB):
                off = pl.multiple_of(rb * GB, GB)
                offh = pl.multiple_of((FB + rb) * GB, GB)
                pltpu.sync_copy(stg.at[pl.ds(off, GB)],
                                bpack_hbm.at[p, s, pl.ds(offh, GB)])
                return 0

            lax.fori_loop(0, nfull, flush, 0)
            base = pl.multiple_of(nfull * GB, GB)
            for j in range(GB // 16):
                stg[pl.ds(j * 16, 16)] = stg[pl.ds(base + j * 16, 16)]
            new += [W2 - base, FB + nfull]
        return tuple(new)

    carry = lax.fori_loop(0, NEB, eblk, (0, 0) * KPC)

    trash = jnp.full((16,), C, jnp.int32)
    for k in range(KPC):
        W, FB = carry[2 * k], carry[2 * k + 1]
        p = k * NC + c
        stg = stgs[k]
        for j in range(4 * GB // 16 + 1):
            stg[pl.ds(W + j * 16, 16)] = trash
        tgt = ((FB * GB + W + 4 * GB - 1) // (4 * GB)) * 4
        nlast = tgt - FB

        def tflush(rb, _, stg=stg, p=p, FB=FB):
            off = pl.multiple_of(rb * GB, GB)
            offh = pl.multiple_of((FB + rb) * GB, GB)
            pltpu.sync_copy(stg.at[pl.ds(off, GB)],
                            bpack_hbm.at[p, s, pl.ds(offh, GB)])
            return 0

        lax.fori_loop(0, nlast, tflush, 0)
        cntbuf[...] = jnp.full((16,), 0, jnp.int32) + tgt
        pltpu.sync_copy(cntbuf.at[pl.ds(0, 8)], bcnt_hbm.at[p, s])


def _agg_body(c, s, x_hbm, bpack_hbm, bcnt_hbm, zeros_hbm, out_hbm,
              pblk, idxs, idxd, rows, acc, cntv, gsem, GBA):
    TB = GBA // GB

    def chunk(k, _):
        p = k * NC + c
        pltpu.sync_copy(bcnt_hbm.at[p, s], cntv.at[pl.ds(0, 8)])
        nblk = cntv[...][0]
        pltpu.sync_copy(zeros_hbm, acc.at[pl.ds(s * STRIPE, STRIPE)])
        plsc.subcore_barrier()

        def blk(t, _):
            pltpu.sync_copy(bpack_hbm.at[p, s, pl.ds(t * GBA, GBA)], pblk)

            def ug(j, _):
                v = pblk[pl.ds(j * 16, 16)]
                idxs[pl.ds(j * 16, 16)] = v >> 13
                idxd[pl.ds(j * 16, 16)] = v & 8191
                return 0

            lax.fori_loop(0, GBA // 16, ug, 0, unroll=4)
            pltpu.async_copy(x_hbm.at[idxs], rows, gsem).wait()
            pltpu.sync_copy(rows, acc.at[idxd], add=True)
            return 0

        lax.fori_loop(0, nblk // TB, blk, 0)
        plsc.subcore_barrier()
        pltpu.sync_copy(
            acc.at[pl.ds(s * STRIPE, STRIPE)],
            out_hbm.at[pl.ds(p * C + s * STRIPE, STRIPE)])
        return 0

    lax.fori_loop(0, KPC, chunk, 0)


GBA_BY_F = {FA: 512, 200: 256, 128: 512}


def _agg_scratch(F):
    GBA = GBA_BY_F[F]
    return [
        pltpu.VMEM((GBA,), jnp.int32),
        pltpu.VMEM((GBA,), jnp.int32),
        pltpu.VMEM((GBA,), jnp.int32),
        pltpu.VMEM((GBA, F), jnp.float32),
        pltpu.VMEM_SHARED((C + 1, F), jnp.float32),
        pltpu.VMEM((16,), jnp.int32),
        pltpu.SemaphoreType.DMA,
    ]


_PART_SCRATCH = [
    pltpu.VMEM((EB,), jnp.int32),
    pltpu.VMEM((EB,), jnp.int32),
    pltpu.VMEM((STG,), jnp.int32),
    pltpu.VMEM((STG,), jnp.int32),
    pltpu.VMEM((STG,), jnp.int32),
    pltpu.VMEM((STG,), jnp.int32),
    pltpu.VMEM((STG,), jnp.int32),
    pltpu.VMEM((16,), jnp.int32),
]


@functools.partial(
    pl.kernel,
    out_type=(jax.ShapeDtypeStruct((NCHUNK, NS, BKT), jnp.int32),
              jax.ShapeDtypeStruct((NCHUNK, NS, 8), jnp.int32),
              jax.ShapeDtypeStruct((NPAD, FA), jnp.float32)),
    mesh=_mesh,
    compiler_params=_sc_params,
    scratch_types=_PART_SCRATCH + _agg_scratch(FA),
)
def _part_agg1(src_hbm, dst_hbm, x_hbm, zeros_hbm,
               bpack_hbm, bcnt_hbm, out_hbm,
               ebs, ebd, stg0, stg1, stg2, stg3, stg4, cntbuf,
               pblk, idxs, idxd, rows, acc, cntv, gsem):
    c = lax.axis_index("c")
    s = lax.axis_index("s")
    _partition_body(c, s, src_hbm, dst_hbm, bpack_hbm, bcnt_hbm,
                    ebs, ebd, (stg0, stg1, stg2, stg3, stg4), cntbuf)
    _agg_body(c, s, x_hbm, bpack_hbm, bcnt_hbm, zeros_hbm, out_hbm,
              pblk, idxs, idxd, rows, acc, cntv, gsem, GBA_BY_F[FA])


def _make_agg(F):
    @functools.partial(
        pl.kernel,
        out_type=jax.ShapeDtypeStruct((NPAD, F), jnp.float32),
        mesh=_mesh,
        compiler_params=_sc_params,
        scratch_types=_agg_scratch(F),
    )
    def agg(x_hbm, bpack_hbm, bcnt_hbm, zeros_hbm, out_hbm,
            pblk, idxs, idxd, rows, acc, cntv, gsem):
        c = lax.axis_index("c")
        s = lax.axis_index("s")
        _agg_body(c, s, x_hbm, bpack_hbm, bcnt_hbm, zeros_hbm, out_hbm,
                  pblk, idxs, idxd, rows, acc, cntv, gsem, GBA_BY_F[F])

    return agg


_agg = {F: _make_agg(F) for F in (FA, 200, 128)}


_RB = 400
_GRID = N // _RB


def _full(shape):
    return pl.BlockSpec(shape, lambda i: (0,) * len(shape))


def _rows(width):
    return pl.BlockSpec((_RB, width), lambda i: (i, 0))


def _clip_aug_body(x_ref, o_ref):
    xb = jnp.clip(x_ref[...], -0.4, 0.4)
    ones = jnp.ones((_RB, 1), jnp.float32)
    zeros = jnp.zeros((_RB, FA - F0 - 1), jnp.float32)
    o_ref[...] = jnp.concatenate([xb, ones, zeros], axis=1)


def _clip_aug(x):
    return pl.pallas_call(
        _clip_aug_body,
        grid=(_GRID,),
        in_specs=[_rows(F0)],
        out_specs=_rows(FA),
        out_shape=jax.ShapeDtypeStruct((N, FA), jnp.float32),
    )(x)


def _matT(a, w):
    return lax.dot_general(a, w, (((1,), (1,)), ((), ())),
                           preferred_element_type=jnp.float32)


def _sage_tail(mean, h, Wl, bl, Wr):
    out = _matT(mean, Wl) + bl + _matT(h, Wr)
    n = jnp.sqrt(jnp.sum(out * out, axis=1, keepdims=True))
    return jnp.maximum(out / jnp.maximum(n, 1e-12), 0.0)


def _layer1_body(sum_ref, xc_ref, Wl_ref, bl_ref, Wr_ref, h_ref, rinv_ref):
    cnt = sum_ref[:, F0:F0 + 1]
    rinv = 1.0 / jnp.maximum(cnt, 1.0)
    mean = sum_ref[:, :F0] * rinv
    h_ref[...] = _sage_tail(mean, xc_ref[:, :F0], Wl_ref[...], bl_ref[...],
                            Wr_ref[...])
    rinv_ref[...] = rinv


def _tc_layer1(sum1, xc, Wl, bl, Wr):
    return pl.pallas_call(
        _layer1_body,
        grid=(_GRID,),
        in_specs=[_rows(FA), _rows(FA), _full(Wl.shape), _full(bl.shape),
                  _full(Wr.shape)],
        out_specs=(_rows(200), _rows(1)),
        out_shape=(jax.ShapeDtypeStruct((N, 200), jnp.float32),
                   jax.ShapeDtypeStruct((N, 1), jnp.float32)),
    )(sum1, xc, Wl, bl, Wr)


def _layer2_body(sum_ref, h_ref, rinv_ref, Wl_ref, bl_ref, Wr_ref, o_ref):
    mean = sum_ref[...] * rinv_ref[...]
    o_ref[...] = _sage_tail(mean, h_ref[...], Wl_ref[...], bl_ref[...],
                            Wr_ref[...])


def _tc_layer2(sum2, h1, rinv, Wl, bl, Wr):
    return pl.pallas_call(
        _layer2_body,
        grid=(_GRID,),
        in_specs=[_rows(200), _rows(200), _rows(1), _full(Wl.shape),
                  _full(bl.shape), _full(Wr.shape)],
        out_specs=_rows(128),
        out_shape=jax.ShapeDtypeStruct((N, 128), jnp.float32),
    )(sum2, h1, rinv, Wl, bl, Wr)


def _layer3_head_body(sum_ref, h_ref, rinv_ref, Wl_ref, bl_ref, Wr_ref,
                      fc1W_ref, fc1b_ref, fc2aW_ref, fc2ab_ref,
                      bng_ref, bnb_ref, fc2bW_ref, fc2bb_ref, o_ref):
    mean = sum_ref[...] * rinv_ref[...]
    h3 = _sage_tail(mean, h_ref[...], Wl_ref[...], bl_ref[...], Wr_ref[...])
    a = jnp.maximum(_matT(h3, fc1W_ref[...]) + fc1b_ref[...], 0.0)
    b = _matT(a, fc2aW_ref[...]) + fc2ab_ref[...]
    b = (b / jnp.sqrt(1.0 + 1e-5)) * bng_ref[...] + bnb_ref[...]
    b = jnp.maximum(b, 0.0)
    lg = _matT(b, fc2bW_ref[...]) + fc2bb_ref[...]
    m = jnp.max(lg, axis=1, keepdims=True)
    ex = jnp.exp(lg - m)
    o_ref[...] = (lg - m) - jnp.log(jnp.sum(ex, axis=1, keepdims=True))


def _tc_layer3_head(sum3, h2, rinv, Wl, bl, Wr, fc1W, fc1b, fc2aW, fc2ab,
                    bng, bnb, fc2bW, fc2bb):
    args = (sum3, h2, rinv, Wl, bl, Wr, fc1W, fc1b, fc2aW, fc2ab, bng, bnb,
            fc2bW, fc2bb)
    in_specs = [_rows(128), _rows(128), _rows(1)] + [
        _full(a.shape) for a in args[3:]]
    return pl.pallas_call(
        _layer3_head_body,
        grid=(_GRID,),
        in_specs=in_specs,
        out_specs=_rows(NLAB),
        out_shape=jax.ShapeDtypeStruct((N, NLAB), jnp.float32),
    )(*args)


def kernel(x, edge_index, Wl1, bl1, Wr1, Wl2, bl2, Wr2, Wl3, bl3, Wr3,
           fc1_W, fc1_b, fc2a_W, fc2a_b, bn_g, bn_b, fc2b_W, fc2b_b):
    src = edge_index[0]
    dst = edge_index[1]
    pad = EPAD - E
    srcp = jnp.concatenate([src, jnp.zeros((pad,), jnp.int32)])
    dstp = jnp.concatenate([dst, jnp.full((pad,), -1, jnp.int32)])

    xc = _clip_aug(x)

    zA = jnp.zeros((STRIPE, FA), jnp.float32)
    z200 = jnp.zeros((STRIPE, 200), jnp.float32)
    z128 = jnp.zeros((STRIPE, 128), jnp.float32)

    bl1r = bl1.reshape(1, -1)
    bl2r = bl2.reshape(1, -1)
    bl3r = bl3.reshape(1, -1)

    bpack, bcnt, sum1p = _part_agg1(srcp, dstp, xc, zA)
    sum1 = sum1p[:N]
    h1, rinv = _tc_layer1(sum1, xc, Wl1, bl1r, Wr1)

    sum2 = _agg[200](h1, bpack, bcnt, z200)[:N]
    h2 = _tc_layer2(sum2, h1, rinv, Wl2, bl2r, Wr2)

    sum3 = _agg[128](h2, bpack, bcnt, z128)[:N]
    out = _tc_layer3_head(
        sum3, h2, rinv, Wl3, bl3r, Wr3,
        fc1_W, fc1_b.reshape(1, -1), fc2a_W, fc2a_b.reshape(1, -1),
        bn_g.reshape(1, -1), bn_b.reshape(1, -1),
        fc2b_W, fc2b_b.reshape(1, -1))
    return out


# speedup vs baseline: 2.2945x; 2.2945x over previous
import functools

import jax
import jax.numpy as jnp
from jax import lax
from jax.experimental import pallas as pl
from jax.experimental.pallas import tpu as pltpu
from jax.experimental.pallas import tpu_sc as plsc

N = 50000
E = 800000
F0 = 100
FA = 104
NLAB = 19

NC = 2
NS = 16
C = 5120
NCHUNK = 10
NPAD = NCHUNK * C
EPT = 51200
EPAD = EPT * NS
EB = 2048
GB = 128
NEB = EPT // EB
NGB = EB // GB
STRIPE = C // NS


SBKT = 26
BKT = SBKT * EB
STG = 2560
KPC = NCHUNK // NC

_mesh = plsc.VectorSubcoreMesh(core_axis_name="c", subcore_axis_name="s")
_sc_params = pltpu.CompilerParams(use_tc_tiling_on_sc=False,
                                  needs_layout_passes=False)


def _partition_body(c, s, src_hbm, dst_hbm, bpack_hbm, bcnt_hbm,
                    ebs, ebd, stgs, cntbuf):
    e0 = s * EPT

    def eblk(b, carry):
        boff = e0 + b * EB
        pltpu.sync_copy(src_hbm.at[pl.ds(boff, EB)], ebs)
        pltpu.sync_copy(dst_hbm.at[pl.ds(boff, EB)], ebd)
        new = []
        for k in range(KPC):
            W, FB = carry[2 * k], carry[2 * k + 1]
            p = k * NC + c
            lo = p * C
            stg = stgs[k]

            def grp(j, w, stg=stg, lo=lo):
                sv = ebs[pl.ds(j * 16, 16)]
                dv = ebd[pl.ds(j * 16, 16)]
                ld = dv - lo
                m = (ld >= 0) & (ld < C)
                packed = sv * 8192 + jnp.where(m, ld, 0)
                _ = plsc.sort_key_val(m.astype(jnp.int32), packed,
                                      descending=True)
                stg[pl.ds(w, 16)] = _[1]
                return w + plsc.all_reduce_population_count(m)[0]

            W2 = lax.fori_loop(0, EB // 16, grp, W, unroll=4)
            nfull = W2 // GB

            def flush(rb, _, stg=stg, p=p, FB=FB):
                off = pl.multiple_of(rb * GB, GB)
                offh = pl.multiple_of((FB + rb) * GB, GB)
                pltpu.sync_copy(stg.at[pl.ds(off, GB)],
                                bpack_hbm.at[p, s, pl.ds(offh, GB)])
                return 0

            lax.fori_loop(0, nfull, flush, 0)
            base = pl.multiple_of(nfull * GB, GB)
            for j in range(GB // 16):
                stg[pl.ds(j * 16, 16)] = stg[pl.ds(base + j * 16, 16)]
            new += [W2 - base, FB + nfull]
        return tuple(new)

    carry = lax.fori_loop(0, NEB, eblk, (0, 0) * KPC)

    trash = jnp.full((16,), C, jnp.int32)
    for k in range(KPC):
        W, FB = carry[2 * k], carry[2 * k + 1]
        p = k * NC + c
        stg = stgs[k]
        for j in range(GB // 16 + 1):
            stg[pl.ds(W + j * 16, 16)] = trash
        tgt = (FB * GB + W + GB - 1) // GB
        nlast = tgt - FB

        def tflush(rb, _, stg=stg, p=p, FB=FB):
            off = pl.multiple_of(rb * GB, GB)
            offh = pl.multiple_of((FB + rb) * GB, GB)
            pltpu.sync_copy(stg.at[pl.ds(off, GB)],
                            bpack_hbm.at[p, s, pl.ds(offh, GB)])
            return 0

        lax.fori_loop(0, nlast, tflush, 0)
        cntbuf[...] = jnp.full((16,), 0, jnp.int32) + tgt
        pltpu.sync_copy(cntbuf.at[pl.ds(0, 8)], bcnt_hbm.at[p, s])


def _agg_body(c, s, x_hbm, bpack_hbm, bcnt_hbm, zeros_hbm, out_hbm,
              pblk, idxs, idxd, rows, acc, cntv, gsem, GBA):
    TB = GBA // GB

    def chunk(k, _):
        p = k * NC + c
        pltpu.sync_copy(bcnt_hbm.at[p, s], cntv.at[pl.ds(0, 8)])
        nblk = cntv[...][0]
        pltpu.sync_copy(zeros_hbm, acc.at[pl.ds(s * STRIPE, STRIPE)])
        plsc.subcore_barrier()

        def blk(t, _):
            pltpu.sync_copy(bpack_hbm.at[p, s, pl.ds(t * GBA, GBA)], pblk)

            def ug(j, _):
                v = pblk[pl.ds(j * 16, 16)]
                idxs[pl.ds(j * 16, 16)] = v >> 13
                idxd[pl.ds(j * 16, 16)] = v & 8191
                return 0

            lax.fori_loop(0, GBA // 16, ug, 0, unroll=4)
            pltpu.async_copy(x_hbm.at[idxs], rows, gsem).wait()
            pltpu.sync_copy(rows, acc.at[idxd], add=True)
            return 0

        lax.fori_loop(0, nblk // TB, blk, 0)
        plsc.subcore_barrier()
        pltpu.sync_copy(
            acc.at[pl.ds(s * STRIPE, STRIPE)],
            out_hbm.at[pl.ds(p * C + s * STRIPE, STRIPE)])
        return 0

    lax.fori_loop(0, KPC, chunk, 0)


GBA_BY_F = {FA: 128, 128: 128}


def _agg_scratch(F):
    GBA = GBA_BY_F[F]
    return [
        pltpu.VMEM((GBA,), jnp.int32),
        pltpu.VMEM((GBA,), jnp.int32),
        pltpu.VMEM((GBA,), jnp.int32),
        pltpu.VMEM((GBA, F), jnp.float32),
        pltpu.VMEM_SHARED((C + 1, F), jnp.float32),
        pltpu.VMEM((16,), jnp.int32),
        pltpu.SemaphoreType.DMA,
    ]


_PART_SCRATCH = [
    pltpu.VMEM((EB,), jnp.int32),
    pltpu.VMEM((EB,), jnp.int32),
    pltpu.VMEM((STG,), jnp.int32),
    pltpu.VMEM((STG,), jnp.int32),
    pltpu.VMEM((STG,), jnp.int32),
    pltpu.VMEM((STG,), jnp.int32),
    pltpu.VMEM((STG,), jnp.int32),
    pltpu.VMEM((16,), jnp.int32),
]


@functools.partial(
    pl.kernel,
    out_type=(jax.ShapeDtypeStruct((NCHUNK, NS, BKT), jnp.int32),
              jax.ShapeDtypeStruct((NCHUNK, NS, 8), jnp.int32),
              jax.ShapeDtypeStruct((NPAD, FA), jnp.float32)),
    mesh=_mesh,
    compiler_params=_sc_params,
    scratch_types=_PART_SCRATCH + _agg_scratch(FA),
)
def _part_agg1(src_hbm, dst_hbm, x_hbm, zeros_hbm,
               bpack_hbm, bcnt_hbm, out_hbm,
               ebs, ebd, stg0, stg1, stg2, stg3, stg4, cntbuf,
               pblk, idxs, idxd, rows, acc, cntv, gsem):
    c = lax.axis_index("c")
    s = lax.axis_index("s")
    _partition_body(c, s, src_hbm, dst_hbm, bpack_hbm, bcnt_hbm,
                    ebs, ebd, (stg0, stg1, stg2, stg3, stg4), cntbuf)
    _agg_body(c, s, x_hbm, bpack_hbm, bcnt_hbm, zeros_hbm, out_hbm,
              pblk, idxs, idxd, rows, acc, cntv, gsem, GBA_BY_F[FA])


def _make_agg(F):
    @functools.partial(
        pl.kernel,
        out_type=jax.ShapeDtypeStruct((NPAD, F), jnp.float32),
        mesh=_mesh,
        compiler_params=_sc_params,
        scratch_types=_agg_scratch(F),
    )
    def agg(x_hbm, bpack_hbm, bcnt_hbm, zeros_hbm, out_hbm,
            pblk, idxs, idxd, rows, acc, cntv, gsem):
        c = lax.axis_index("c")
        s = lax.axis_index("s")
        _agg_body(c, s, x_hbm, bpack_hbm, bcnt_hbm, zeros_hbm, out_hbm,
                  pblk, idxs, idxd, rows, acc, cntv, gsem, GBA_BY_F[F])

    return agg


_agg = {F: _make_agg(F) for F in (128,)}


_RB = 400
_GRID = N // _RB


def _full(shape):
    return pl.BlockSpec(shape, lambda i: (0,) * len(shape))


def _rows(width):
    return pl.BlockSpec((_RB, width), lambda i: (i, 0))


def _clip_aug_body(x_ref, o_ref):
    xb = jnp.clip(x_ref[...], -0.4, 0.4)
    ones = jnp.ones((_RB, 1), jnp.float32)
    zeros = jnp.zeros((_RB, FA - F0 - 1), jnp.float32)
    o_ref[...] = jnp.concatenate([xb, ones, zeros], axis=1)


def _clip_aug(x):
    return pl.pallas_call(
        _clip_aug_body,
        grid=(_GRID,),
        in_specs=[_rows(F0)],
        out_specs=_rows(FA),
        out_shape=jax.ShapeDtypeStruct((N, FA), jnp.float32),
    )(x)


def _matT(a, w):
    return lax.dot_general(a, w, (((1,), (1,)), ((), ())),
                           preferred_element_type=jnp.float32)


def _sage_tail(mean, h, Wl, bl, Wr):
    out = _matT(mean, Wl) + bl + _matT(h, Wr)
    n = jnp.sqrt(jnp.sum(out * out, axis=1, keepdims=True))
    return jnp.maximum(out / jnp.maximum(n, 1e-12), 0.0)


def _layer1_body(sum_ref, xc_ref, Wl_ref, bl_ref, Wr_ref, Wl2_ref,
                 h_ref, rinv_ref, pre2_ref):
    cnt = sum_ref[:, F0:F0 + 1]
    rinv = 1.0 / jnp.maximum(cnt, 1.0)
    mean = sum_ref[:, :F0] * rinv
    h = _sage_tail(mean, xc_ref[:, :F0], Wl_ref[...], bl_ref[...],
                   Wr_ref[...])
    h_ref[...] = h
    rinv_ref[...] = rinv
    pre2_ref[...] = _matT(h, Wl2_ref[...])


def _tc_layer1(sum1, xc, Wl, bl, Wr, Wl2):
    return pl.pallas_call(
        _layer1_body,
        grid=(_GRID,),
        in_specs=[_rows(FA), _rows(FA), _full(Wl.shape), _full(bl.shape),
                  _full(Wr.shape), _full(Wl2.shape)],
        out_specs=(_rows(200), _rows(1), _rows(128)),
        out_shape=(jax.ShapeDtypeStruct((N, 200), jnp.float32),
                   jax.ShapeDtypeStruct((N, 1), jnp.float32),
                   jax.ShapeDtypeStruct((N, 128), jnp.float32)),
    )(sum1, xc, Wl, bl, Wr, Wl2)


def _layer2_body(sum_ref, h_ref, rinv_ref, bl_ref, Wr_ref, o_ref):
    meanW = sum_ref[...] * rinv_ref[...]
    out = meanW + bl_ref[...] + _matT(h_ref[...], Wr_ref[...])
    n = jnp.sqrt(jnp.sum(out * out, axis=1, keepdims=True))
    o_ref[...] = jnp.maximum(out / jnp.maximum(n, 1e-12), 0.0)


def _tc_layer2(sum2, h1, rinv, bl, Wr):
    return pl.pallas_call(
        _layer2_body,
        grid=(_GRID,),
        in_specs=[_rows(128), _rows(200), _rows(1),
                  _full(bl.shape), _full(Wr.shape)],
        out_specs=_rows(128),
        out_shape=jax.ShapeDtypeStruct((N, 128), jnp.float32),
    )(sum2, h1, rinv, bl, Wr)


def _layer3_head_body(sum_ref, h_ref, rinv_ref, Wl_ref, bl_ref, Wr_ref,
                      fc1W_ref, fc1b_ref, fc2aW_ref, fc2ab_ref,
                      bng_ref, bnb_ref, fc2bW_ref, fc2bb_ref, o_ref):
    mean = sum_ref[...] * rinv_ref[...]
    h3 = _sage_tail(mean, h_ref[...], Wl_ref[...], bl_ref[...], Wr_ref[...])
    a = jnp.maximum(_matT(h3, fc1W_ref[...]) + fc1b_ref[...], 0.0)
    b = _matT(a, fc2aW_ref[...]) + fc2ab_ref[...]
    b = (b / jnp.sqrt(1.0 + 1e-5)) * bng_ref[...] + bnb_ref[...]
    b = jnp.maximum(b, 0.0)
    lg = _matT(b, fc2bW_ref[...]) + fc2bb_ref[...]
    m = jnp.max(lg, axis=1, keepdims=True)
    ex = jnp.exp(lg - m)
    o_ref[...] = (lg - m) - jnp.log(jnp.sum(ex, axis=1, keepdims=True))


def _tc_layer3_head(sum3, h2, rinv, Wl, bl, Wr, fc1W, fc1b, fc2aW, fc2ab,
                    bng, bnb, fc2bW, fc2bb):
    args = (sum3, h2, rinv, Wl, bl, Wr, fc1W, fc1b, fc2aW, fc2ab, bng, bnb,
            fc2bW, fc2bb)
    in_specs = [_rows(128), _rows(128), _rows(1)] + [
        _full(a.shape) for a in args[3:]]
    return pl.pallas_call(
        _layer3_head_body,
        grid=(_GRID,),
        in_specs=in_specs,
        out_specs=_rows(NLAB),
        out_shape=jax.ShapeDtypeStruct((N, NLAB), jnp.float32),
    )(*args)


def kernel(x, edge_index, Wl1, bl1, Wr1, Wl2, bl2, Wr2, Wl3, bl3, Wr3,
           fc1_W, fc1_b, fc2a_W, fc2a_b, bn_g, bn_b, fc2b_W, fc2b_b):
    src = edge_index[0]
    dst = edge_index[1]
    pad = EPAD - E
    srcp = jnp.concatenate([src, jnp.zeros((pad,), jnp.int32)])
    dstp = jnp.concatenate([dst, jnp.full((pad,), -1, jnp.int32)])

    xc = _clip_aug(x)

    zA = jnp.zeros((STRIPE, FA), jnp.float32)
    z128 = jnp.zeros((STRIPE, 128), jnp.float32)

    bl1r = bl1.reshape(1, -1)
    bl2r = bl2.reshape(1, -1)
    bl3r = bl3.reshape(1, -1)

    bpack, bcnt, sum1p = _part_agg1(srcp, dstp, xc, zA)
    sum1 = sum1p[:N]
    h1, rinv, pre2 = _tc_layer1(sum1, xc, Wl1, bl1r, Wr1, Wl2)

    sum2 = _agg[128](pre2, bpack, bcnt, z128)[:N]
    h2 = _tc_layer2(sum2, h1, rinv, bl2r, Wr2)

    sum3 = _agg[128](h2, bpack, bcnt, z128)[:N]
    out = _tc_layer3_head(
        sum3, h2, rinv, Wl3, bl3r, Wr3,
        fc1_W, fc1_b.reshape(1, -1), fc2a_W, fc2a_b.reshape(1, -1),
        bn_g.reshape(1, -1), bn_b.reshape(1, -1),
        fc2b_W, fc2b_b.reshape(1, -1))
    return out


# speedup vs baseline: 2.7070x; 1.1798x over previous
import functools

import jax
import jax.numpy as jnp
from jax import lax
from jax.experimental import pallas as pl
from jax.experimental.pallas import tpu as pltpu
from jax.experimental.pallas import tpu_sc as plsc

N = 50000
E = 800000
F0 = 100
FA = 104
NLAB = 19

NC = 2
NS = 16
C = 6400
NCHUNK = 8
NPAD = NCHUNK * C
EPT = 51200
EPAD = EPT * NS
EB = 2048
GB = 128
NEB = EPT // EB
NGB = EB // GB
STRIPE = C // NS


SBKT = 26
BKT = SBKT * EB
STG = 2560
KPC = NCHUNK // NC

_mesh = plsc.VectorSubcoreMesh(core_axis_name="c", subcore_axis_name="s")
_sc_params = pltpu.CompilerParams(use_tc_tiling_on_sc=False,
                                  needs_layout_passes=False)


def _partition_body(c, s, src_hbm, dst_hbm, bpack_hbm, bcnt_hbm,
                    ebs, ebd, stgs, cntbuf):
    e0 = s * EPT

    def eblk(b, carry):
        boff = e0 + b * EB
        pltpu.sync_copy(src_hbm.at[pl.ds(boff, EB)], ebs)
        pltpu.sync_copy(dst_hbm.at[pl.ds(boff, EB)], ebd)
        new = []
        for k in range(KPC):
            W, FB = carry[2 * k], carry[2 * k + 1]
            p = k * NC + c
            lo = p * C
            stg = stgs[k]

            def grp(j, w, stg=stg, lo=lo):
                sv = ebs[pl.ds(j * 16, 16)]
                dv = ebd[pl.ds(j * 16, 16)]
                ld = dv - lo
                m = (ld >= 0) & (ld < C)
                packed = sv * 8192 + jnp.where(m, ld, 0)
                _ = plsc.sort_key_val(m.astype(jnp.int32), packed,
                                      descending=True)
                stg[pl.ds(w, 16)] = _[1]
                return w + plsc.all_reduce_population_count(m)[0]

            W2 = lax.fori_loop(0, EB // 16, grp, W, unroll=4)
            nfull = W2 // GB

            def flush(rb, _, stg=stg, p=p, FB=FB):
                off = pl.multiple_of(rb * GB, GB)
                offh = pl.multiple_of((FB + rb) * GB, GB)
                pltpu.sync_copy(stg.at[pl.ds(off, GB)],
                                bpack_hbm.at[p, s, pl.ds(offh, GB)])
                return 0

            lax.fori_loop(0, nfull, flush, 0)
            base = pl.multiple_of(nfull * GB, GB)
            for j in range(GB // 16):
                stg[pl.ds(j * 16, 16)] = stg[pl.ds(base + j * 16, 16)]
            new += [W2 - base, FB + nfull]
        return tuple(new)

    carry = lax.fori_loop(0, NEB, eblk, (0, 0) * KPC)

    trash = jnp.full((16,), C, jnp.int32)
    for k in range(KPC):
        W, FB = carry[2 * k], carry[2 * k + 1]
        p = k * NC + c
        stg = stgs[k]
        for j in range(GB // 16 + 1):
            stg[pl.ds(W + j * 16, 16)] = trash
        tgt = (FB * GB + W + GB - 1) // GB
        nlast = tgt - FB

        def tflush(rb, _, stg=stg, p=p, FB=FB):
            off = pl.multiple_of(rb * GB, GB)
            offh = pl.multiple_of((FB + rb) * GB, GB)
            pltpu.sync_copy(stg.at[pl.ds(off, GB)],
                            bpack_hbm.at[p, s, pl.ds(offh, GB)])
            return 0

        lax.fori_loop(0, nlast, tflush, 0)
        cntbuf[...] = jnp.full((16,), 0, jnp.int32) + tgt
        pltpu.sync_copy(cntbuf.at[pl.ds(0, 8)], bcnt_hbm.at[p, s])


def _agg_body(c, s, x_hbm, bpack_hbm, bcnt_hbm, zeros_hbm, out_hbm,
              pbuf, idxs, idxd, rows, acc, cntv, gsem):
    def chunk(k, _):
        p = k * NC + c
        pltpu.sync_copy(bcnt_hbm.at[p, s], cntv.at[pl.ds(0, 8)])
        nblk = cntv[...][0]
        pltpu.sync_copy(zeros_hbm, acc.at[pl.ds(s * STRIPE, STRIPE)])
        plsc.subcore_barrier()

        def blk(b, _):
            @pl.when(lax.rem(b, 16) == 0)
            def _():
                pltpu.sync_copy(
                    bpack_hbm.at[p, s, pl.ds((b // 16) * EB, EB)], pbuf)
            off = lax.rem(b, 16) * GB

            def ug(j, _):
                v = pbuf[pl.ds(off + j * 16, 16)]
                idxs[pl.ds(j * 16, 16)] = v >> 13
                idxd[pl.ds(j * 16, 16)] = v & 8191
                return 0

            lax.fori_loop(0, GB // 16, ug, 0, unroll=8)
            pltpu.async_copy(x_hbm.at[idxs], rows, gsem).wait()
            pltpu.sync_copy(rows, acc.at[idxd], add=True)
            return 0

        lax.fori_loop(0, nblk, blk, 0)
        plsc.subcore_barrier()
        pltpu.sync_copy(
            acc.at[pl.ds(s * STRIPE, STRIPE)],
            out_hbm.at[pl.ds(p * C + s * STRIPE, STRIPE)])
        return 0

    lax.fori_loop(0, KPC, chunk, 0)


def _agg_scratch(F):
    return [
        pltpu.VMEM((EB,), jnp.int32),
        pltpu.VMEM((GB,), jnp.int32),
        pltpu.VMEM((GB,), jnp.int32),
        pltpu.VMEM((GB, F), jnp.float32),
        pltpu.VMEM_SHARED((C + 1, F), jnp.float32),
        pltpu.VMEM((16,), jnp.int32),
        pltpu.SemaphoreType.DMA,
    ]


_PART_SCRATCH = [
    pltpu.VMEM((EB,), jnp.int32),
    pltpu.VMEM((EB,), jnp.int32),
    pltpu.VMEM((STG,), jnp.int32),
    pltpu.VMEM((STG,), jnp.int32),
    pltpu.VMEM((STG,), jnp.int32),
    pltpu.VMEM((STG,), jnp.int32),
    pltpu.VMEM((16,), jnp.int32),
]


@functools.partial(
    pl.kernel,
    out_type=(jax.ShapeDtypeStruct((NCHUNK, NS, BKT), jnp.int32),
              jax.ShapeDtypeStruct((NCHUNK, NS, 8), jnp.int32),
              jax.ShapeDtypeStruct((NPAD, FA), jnp.float32)),
    mesh=_mesh,
    compiler_params=_sc_params,
    scratch_types=_PART_SCRATCH + _agg_scratch(FA),
)
def _part_agg1(src_hbm, dst_hbm, x_hbm, zeros_hbm,
               bpack_hbm, bcnt_hbm, out_hbm,
               ebs, ebd, stg0, stg1, stg2, stg3, cntbuf,
               pbuf, idxs, idxd, rows, acc, cntv, gsem):
    c = lax.axis_index("c")
    s = lax.axis_index("s")
    _partition_body(c, s, src_hbm, dst_hbm, bpack_hbm, bcnt_hbm,
                    ebs, ebd, (stg0, stg1, stg2, stg3), cntbuf)
    _agg_body(c, s, x_hbm, bpack_hbm, bcnt_hbm, zeros_hbm, out_hbm,
              pbuf, idxs, idxd, rows, acc, cntv, gsem)


def _make_agg(F):
    @functools.partial(
        pl.kernel,
        out_type=jax.ShapeDtypeStruct((NPAD, F), jnp.float32),
        mesh=_mesh,
        compiler_params=_sc_params,
        scratch_types=_agg_scratch(F),
    )
    def agg(x_hbm, bpack_hbm, bcnt_hbm, zeros_hbm, out_hbm,
            pbuf, idxs, idxd, rows, acc, cntv, gsem):
        c = lax.axis_index("c")
        s = lax.axis_index("s")
        _agg_body(c, s, x_hbm, bpack_hbm, bcnt_hbm, zeros_hbm, out_hbm,
                  pbuf, idxs, idxd, rows, acc, cntv, gsem)

    return agg


_agg = {F: _make_agg(F) for F in (128,)}


_RB = 400
_GRID = N // _RB


def _full(shape):
    return pl.BlockSpec(shape, lambda i: (0,) * len(shape))


def _rows(width):
    return pl.BlockSpec((_RB, width), lambda i: (i, 0))


def _clip_aug_body(x_ref, o_ref):
    xb = jnp.clip(x_ref[...], -0.4, 0.4)
    ones = jnp.ones((_RB, 1), jnp.float32)
    zeros = jnp.zeros((_RB, FA - F0 - 1), jnp.float32)
    o_ref[...] = jnp.concatenate([xb, ones, zeros], axis=1)


def _clip_aug(x):
    return pl.pallas_call(
        _clip_aug_body,
        grid=(_GRID,),
        in_specs=[_rows(F0)],
        out_specs=_rows(FA),
        out_shape=jax.ShapeDtypeStruct((N, FA), jnp.float32),
    )(x)


def _matT(a, w):
    return lax.dot_general(a, w, (((1,), (1,)), ((), ())),
                           preferred_element_type=jnp.float32)


def _sage_tail(mean, h, Wl, bl, Wr):
    out = _matT(mean, Wl) + bl + _matT(h, Wr)
    n = jnp.sqrt(jnp.sum(out * out, axis=1, keepdims=True))
    return jnp.maximum(out / jnp.maximum(n, 1e-12), 0.0)


def _layer1_body(sum_ref, xc_ref, Wl_ref, bl_ref, Wr_ref, Wl2_ref,
                 h_ref, rinv_ref, pre2_ref):
    cnt = sum_ref[:, F0:F0 + 1]
    rinv = 1.0 / jnp.maximum(cnt, 1.0)
    mean = sum_ref[:, :F0] * rinv
    h = _sage_tail(mean, xc_ref[:, :F0], Wl_ref[...], bl_ref[...],
                   Wr_ref[...])
    h_ref[...] = h
    rinv_ref[...] = rinv
    pre2_ref[...] = _matT(h, Wl2_ref[...])


def _tc_layer1(sum1, xc, Wl, bl, Wr, Wl2):
    return pl.pallas_call(
        _layer1_body,
        grid=(_GRID,),
        in_specs=[_rows(FA), _rows(FA), _full(Wl.shape), _full(bl.shape),
                  _full(Wr.shape), _full(Wl2.shape)],
        out_specs=(_rows(200), _rows(1), _rows(128)),
        out_shape=(jax.ShapeDtypeStruct((N, 200), jnp.float32),
                   jax.ShapeDtypeStruct((N, 1), jnp.float32),
                   jax.ShapeDtypeStruct((N, 128), jnp.float32)),
    )(sum1, xc, Wl, bl, Wr, Wl2)


def _layer2_body(sum_ref, h_ref, rinv_ref, bl_ref, Wr_ref, o_ref):
    meanW = sum_ref[...] * rinv_ref[...]
    out = meanW + bl_ref[...] + _matT(h_ref[...], Wr_ref[...])
    n = jnp.sqrt(jnp.sum(out * out, axis=1, keepdims=True))
    o_ref[...] = jnp.maximum(out / jnp.maximum(n, 1e-12), 0.0)


def _tc_layer2(sum2, h1, rinv, bl, Wr):
    return pl.pallas_call(
        _layer2_body,
        grid=(_GRID,),
        in_specs=[_rows(128), _rows(200), _rows(1),
                  _full(bl.shape), _full(Wr.shape)],
        out_specs=_rows(128),
        out_shape=jax.ShapeDtypeStruct((N, 128), jnp.float32),
    )(sum2, h1, rinv, bl, Wr)


def _layer3_head_body(sum_ref, h_ref, rinv_ref, Wl_ref, bl_ref, Wr_ref,
                      fc1W_ref, fc1b_ref, fc2aW_ref, fc2ab_ref,
                      bng_ref, bnb_ref, fc2bW_ref, fc2bb_ref, o_ref):
    mean = sum_ref[...] * rinv_ref[...]
    h3 = _sage_tail(mean, h_ref[...], Wl_ref[...], bl_ref[...], Wr_ref[...])
    a = jnp.maximum(_matT(h3, fc1W_ref[...]) + fc1b_ref[...], 0.0)
    b = _matT(a, fc2aW_ref[...]) + fc2ab_ref[...]
    b = (b / jnp.sqrt(1.0 + 1e-5)) * bng_ref[...] + bnb_ref[...]
    b = jnp.maximum(b, 0.0)
    lg = _matT(b, fc2bW_ref[...]) + fc2bb_ref[...]
    m = jnp.max(lg, axis=1, keepdims=True)
    ex = jnp.exp(lg - m)
    o_ref[...] = (lg - m) - jnp.log(jnp.sum(ex, axis=1, keepdims=True))


def _tc_layer3_head(sum3, h2, rinv, Wl, bl, Wr, fc1W, fc1b, fc2aW, fc2ab,
                    bng, bnb, fc2bW, fc2bb):
    args = (sum3, h2, rinv, Wl, bl, Wr, fc1W, fc1b, fc2aW, fc2ab, bng, bnb,
            fc2bW, fc2bb)
    in_specs = [_rows(128), _rows(128), _rows(1)] + [
        _full(a.shape) for a in args[3:]]
    return pl.pallas_call(
        _layer3_head_body,
        grid=(_GRID,),
        in_specs=in_specs,
        out_specs=_rows(NLAB),
        out_shape=jax.ShapeDtypeStruct((N, NLAB), jnp.float32),
    )(*args)


def kernel(x, edge_index, Wl1, bl1, Wr1, Wl2, bl2, Wr2, Wl3, bl3, Wr3,
           fc1_W, fc1_b, fc2a_W, fc2a_b, bn_g, bn_b, fc2b_W, fc2b_b):
    src = edge_index[0]
    dst = edge_index[1]
    pad = EPAD - E
    srcp = jnp.concatenate([src, jnp.zeros((pad,), jnp.int32)])
    dstp = jnp.concatenate([dst, jnp.full((pad,), -1, jnp.int32)])

    xc = _clip_aug(x)

    zA = jnp.zeros((STRIPE, FA), jnp.float32)
    z128 = jnp.zeros((STRIPE, 128), jnp.float32)

    bl1r = bl1.reshape(1, -1)
    bl2r = bl2.reshape(1, -1)
    bl3r = bl3.reshape(1, -1)

    bpack, bcnt, sum1p = _part_agg1(srcp, dstp, xc, zA)
    sum1 = sum1p[:N]
    h1, rinv, pre2 = _tc_layer1(sum1, xc, Wl1, bl1r, Wr1, Wl2)

    sum2 = _agg[128](pre2, bpack, bcnt, z128)[:N]
    h2 = _tc_layer2(sum2, h1, rinv, bl2r, Wr2)

    sum3 = _agg[128](h2, bpack, bcnt, z128)[:N]
    out = _tc_layer3_head(
        sum3, h2, rinv, Wl3, bl3r, Wr3,
        fc1_W, fc1_b.reshape(1, -1), fc2a_W, fc2a_b.reshape(1, -1),
        bn_g.reshape(1, -1), bn_b.reshape(1, -1),
        fc2b_W, fc2b_b.reshape(1, -1))
    return out
